# speedup vs baseline: 2.9168x; 2.9168x over previous
import functools

import jax
import jax.numpy as jnp
from jax import lax
from jax.experimental import pallas as pl
from jax.experimental.pallas import tpu as pltpu
from jax.experimental.pallas import tpu_sc as plsc

D_MODEL = 128
NUM_CORES = 2
NUM_SUBCORES = 16
NUM_WORKERS = NUM_CORES * NUM_SUBCORES

ROWS_PER_STREAM = 128
STREAMS_PER_GROUP = 2
GROUP = ROWS_PER_STREAM * STREAMS_PER_GROUP


def _make_gather(B: int):
  assert B % (NUM_WORKERS * GROUP) == 0
  b_per_w = B // NUM_WORKERS
  n_groups = b_per_w // GROUP

  mesh = plsc.VectorSubcoreMesh(
      core_axis_name="c", subcore_axis_name="s",
      num_cores=NUM_CORES, num_subcores=NUM_SUBCORES)

  @functools.partial(
      pl.kernel,
      out_type=jax.ShapeDtypeStruct((B, D_MODEL), jnp.float32),
      mesh=mesh,
      scratch_types=[
          pltpu.VMEM((STREAMS_PER_GROUP, ROWS_PER_STREAM), jnp.int32),
          pltpu.VMEM((GROUP, D_MODEL), jnp.float32),
          pltpu.SemaphoreType.DMA,
      ],
  )
  def gather_kernel(idx_hbm, table_hbm, out_hbm, idx_v, rows_v, gsem):
    wid = lax.axis_index("s") * NUM_CORES + lax.axis_index("c")
    base_row = wid * (b_per_w // ROWS_PER_STREAM)

    @pl.loop(0, n_groups)
    def _(g):
      row0 = base_row + g * STREAMS_PER_GROUP
      pltpu.sync_copy(idx_hbm.at[pl.ds(row0, STREAMS_PER_GROUP)], idx_v)
      copies = []
      for j in range(STREAMS_PER_GROUP):
        copies.append(pltpu.async_copy(
            table_hbm.at[idx_v.at[j]],
            rows_v.at[pl.ds(j * ROWS_PER_STREAM, ROWS_PER_STREAM)],
            gsem))
      for c in copies:
        c.wait()
      out0 = wid * b_per_w + g * GROUP
      pltpu.sync_copy(rows_v, out_hbm.at[pl.ds(out0, GROUP)])

  return gather_kernel


def kernel(var_indices, var_embedding):
  n, s = var_indices.shape
  B = n * s
  idx2d = var_indices.reshape(B // ROWS_PER_STREAM, ROWS_PER_STREAM)
  idx2d = idx2d.astype(jnp.int32)
  out = _make_gather(B)(idx2d, var_embedding)
  return out.reshape(n, s, var_embedding.shape[1])


# speedup vs baseline: 2.9537x; 1.0127x over previous
import functools

import jax
import jax.numpy as jnp
from jax import lax
from jax.experimental import pallas as pl
from jax.experimental.pallas import tpu as pltpu
from jax.experimental.pallas import tpu_sc as plsc

D_MODEL = 128
NUM_CORES = 2
NUM_SUBCORES = 16
NUM_WORKERS = NUM_CORES * NUM_SUBCORES

ROWS_PER_STREAM = 128
STREAMS_PER_GROUP = 2
GROUP = ROWS_PER_STREAM * STREAMS_PER_GROUP


def _make_gather(B: int):
  assert B % (NUM_WORKERS * GROUP) == 0
  b_per_w = B // NUM_WORKERS
  n_groups = b_per_w // GROUP
  assert n_groups % 2 == 0

  mesh = plsc.VectorSubcoreMesh(
      core_axis_name="c", subcore_axis_name="s",
      num_cores=NUM_CORES, num_subcores=NUM_SUBCORES)

  @functools.partial(
      pl.kernel,
      out_type=jax.ShapeDtypeStruct((B, D_MODEL), jnp.float32),
      mesh=mesh,
      scratch_types=[
          pltpu.VMEM((2, STREAMS_PER_GROUP, ROWS_PER_STREAM), jnp.int32),
          pltpu.VMEM((2, GROUP, D_MODEL), jnp.float32),
          pltpu.SemaphoreType.DMA,
          pltpu.SemaphoreType.DMA,
          pltpu.SemaphoreType.DMA,
          pltpu.SemaphoreType.DMA,
      ],
  )
  def gather_kernel(idx_hbm, table_hbm, out_hbm, idx_v, rows_v,
                    gsem0, gsem1, ssem0, ssem1):
    wid = lax.axis_index("s") * NUM_CORES + lax.axis_index("c")
    base_row = wid * (b_per_w // ROWS_PER_STREAM)
    out_base = wid * b_per_w
    gsem = (gsem0, gsem1)
    ssem = (ssem0, ssem1)

    def fire_gather(g, buf):
      row0 = base_row + g * STREAMS_PER_GROUP
      pltpu.sync_copy(idx_hbm.at[pl.ds(row0, STREAMS_PER_GROUP)],
                      idx_v.at[buf])
      for j in range(STREAMS_PER_GROUP):
        pltpu.async_copy(
            table_hbm.at[idx_v.at[buf].at[j]],
            rows_v.at[buf].at[pl.ds(j * ROWS_PER_STREAM, ROWS_PER_STREAM)],
            gsem[buf])

    def wait_gather(buf):
      for j in range(STREAMS_PER_GROUP):
        pltpu.make_async_copy(
            table_hbm.at[idx_v.at[buf].at[j]],
            rows_v.at[buf].at[pl.ds(j * ROWS_PER_STREAM, ROWS_PER_STREAM)],
            gsem[buf]).wait()

    def wait_scatter(buf):
      pltpu.make_async_copy(
          rows_v.at[buf], out_hbm.at[pl.ds(out_base, GROUP)],
          ssem[buf]).wait()

    fire_gather(0, 0)

    @pl.loop(0, n_groups // 2)
    def _(p):
      for buf in (0, 1):
        g = 2 * p + buf
        other = 1 - buf

        @pl.when(g + 1 < n_groups)
        def _():
          @pl.when(g >= 1)
          def _():
            wait_scatter(other)
          fire_gather(g + 1, other)

        wait_gather(buf)
        pltpu.async_copy(
            rows_v.at[buf],
            out_hbm.at[pl.ds(out_base + g * GROUP, GROUP)],
            ssem[buf])

    wait_scatter(0)
    wait_scatter(1)

  return gather_kernel


def kernel(var_indices, var_embedding):
  n, s = var_indices.shape
  B = n * s
  idx2d = var_indices.reshape(B // ROWS_PER_STREAM, ROWS_PER_STREAM)
  idx2d = idx2d.astype(jnp.int32)
  out = _make_gather(B)(idx2d, var_embedding)
  return out.reshape(n, s, var_embedding.shape[1])


# speedup vs baseline: 4.9248x; 1.6673x over previous
import functools

import jax
import jax.numpy as jnp
from jax import lax
from jax.experimental import pallas as pl
from jax.experimental.pallas import tpu as pltpu
from jax.experimental.pallas import tpu_sc as plsc

D_MODEL = 128
NUM_CORES = 2
NUM_SUBCORES = 16
NUM_WORKERS = NUM_CORES * NUM_SUBCORES

TOKENS_PER_GROUP = 4


def _make_gather(N: int, S: int):
  assert N % (NUM_WORKERS * TOKENS_PER_GROUP) == 0
  t_per_w = N // NUM_WORKERS
  n_groups = t_per_w // TOKENS_PER_GROUP
  assert n_groups % 2 == 0

  mesh = plsc.VectorSubcoreMesh(
      core_axis_name="c", subcore_axis_name="s",
      num_cores=NUM_CORES, num_subcores=NUM_SUBCORES)

  @functools.partial(
      pl.kernel,
      out_type=jax.ShapeDtypeStruct((N, S, D_MODEL), jnp.float32),
      mesh=mesh,
      scratch_types=[
          pltpu.VMEM((2, TOKENS_PER_GROUP, S), jnp.int32),
          pltpu.VMEM((2, TOKENS_PER_GROUP, S, D_MODEL), jnp.float32),
          pltpu.SemaphoreType.DMA,
          pltpu.SemaphoreType.DMA,
          pltpu.SemaphoreType.DMA,
          pltpu.SemaphoreType.DMA,
      ],
  )
  def gather_kernel(idx_hbm, table_hbm, out_hbm, idx_v, rows_v,
                    gsem0, gsem1, ssem0, ssem1):
    wid = lax.axis_index("s") * NUM_CORES + lax.axis_index("c")
    tok_base = wid * t_per_w
    gsem = (gsem0, gsem1)
    ssem = (ssem0, ssem1)

    def fire_gather(g, buf):
      tok0 = tok_base + g * TOKENS_PER_GROUP
      pltpu.sync_copy(idx_hbm.at[pl.ds(tok0, TOKENS_PER_GROUP)],
                      idx_v.at[buf])
      for j in range(TOKENS_PER_GROUP):
        pltpu.async_copy(
            table_hbm.at[idx_v.at[buf].at[j]],
            rows_v.at[buf].at[j],
            gsem[buf])

    def wait_gather(buf):
      for j in range(TOKENS_PER_GROUP):
        pltpu.make_async_copy(
            table_hbm.at[idx_v.at[buf].at[j]],
            rows_v.at[buf].at[j],
            gsem[buf]).wait()

    def wait_scatter(buf):
      pltpu.make_async_copy(
          rows_v.at[buf], out_hbm.at[pl.ds(tok_base, TOKENS_PER_GROUP)],
          ssem[buf]).wait()

    fire_gather(0, 0)

    @pl.loop(0, n_groups // 2)
    def _(p):
      for buf in (0, 1):
        g = 2 * p + buf
        other = 1 - buf

        @pl.when(g + 1 < n_groups)
        def _():
          @pl.when(g >= 1)
          def _():
            wait_scatter(other)
          fire_gather(g + 1, other)

        wait_gather(buf)
        pltpu.async_copy(
            rows_v.at[buf],
            out_hbm.at[pl.ds(tok_base + g * TOKENS_PER_GROUP,
                             TOKENS_PER_GROUP)],
            ssem[buf])

    wait_scatter(0)
    wait_scatter(1)

  return gather_kernel


def kernel(var_indices, var_embedding):
  n, s = var_indices.shape
  idx = var_indices.astype(jnp.int32)
  return _make_gather(n, s)(idx, var_embedding)


# speedup vs baseline: 4.9286x; 1.0008x over previous
import functools

import jax
import jax.numpy as jnp
from jax import lax
from jax.experimental import pallas as pl
from jax.experimental.pallas import tpu as pltpu
from jax.experimental.pallas import tpu_sc as plsc

D_MODEL = 128
NUM_CORES = 2
NUM_SUBCORES = 16
NUM_WORKERS = NUM_CORES * NUM_SUBCORES

TOKENS_PER_GROUP = 4


def _make_gather(N: int, S: int):
  assert N % (NUM_WORKERS * TOKENS_PER_GROUP) == 0
  t_per_w = N // NUM_WORKERS
  n_groups = t_per_w // TOKENS_PER_GROUP
  assert n_groups % 2 == 0

  mesh = plsc.VectorSubcoreMesh(
      core_axis_name="c", subcore_axis_name="s",
      num_cores=NUM_CORES, num_subcores=NUM_SUBCORES)

  @functools.partial(
      pl.kernel,
      out_type=jax.ShapeDtypeStruct((N, S, D_MODEL), jnp.float32),
      mesh=mesh,
      compiler_params=pltpu.CompilerParams(use_tc_tiling_on_sc=True),
      scratch_types=[
          pltpu.VMEM((2, TOKENS_PER_GROUP, S), jnp.int32),
          pltpu.VMEM((2, TOKENS_PER_GROUP, S, D_MODEL), jnp.float32),
          pltpu.SemaphoreType.DMA,
          pltpu.SemaphoreType.DMA,
          pltpu.SemaphoreType.DMA,
          pltpu.SemaphoreType.DMA,
      ],
  )
  def gather_kernel(idx_hbm, table_hbm, out_hbm, idx_v, rows_v,
                    gsem0, gsem1, ssem0, ssem1):
    wid = lax.axis_index("s") * NUM_CORES + lax.axis_index("c")
    tok_base = wid * t_per_w
    gsem = (gsem0, gsem1)
    ssem = (ssem0, ssem1)

    def fire_gather(g, buf):
      tok0 = tok_base + g * TOKENS_PER_GROUP
      pltpu.sync_copy(idx_hbm.at[pl.ds(tok0, TOKENS_PER_GROUP)],
                      idx_v.at[buf])
      for j in range(TOKENS_PER_GROUP):
        pltpu.async_copy(
            table_hbm.at[idx_v.at[buf].at[j]],
            rows_v.at[buf].at[j],
            gsem[buf])

    def wait_gather(buf):
      for j in range(TOKENS_PER_GROUP):
        pltpu.make_async_copy(
            table_hbm.at[idx_v.at[buf].at[j]],
            rows_v.at[buf].at[j],
            gsem[buf]).wait()

    def wait_scatter(buf):
      pltpu.make_async_copy(
          rows_v.at[buf], out_hbm.at[pl.ds(tok_base, TOKENS_PER_GROUP)],
          ssem[buf]).wait()

    fire_gather(0, 0)

    @pl.loop(0, n_groups // 2)
    def _(p):
      for buf in (0, 1):
        g = 2 * p + buf
        other = 1 - buf

        @pl.when(g + 1 < n_groups)
        def _():
          @pl.when(g >= 1)
          def _():
            wait_scatter(other)
          fire_gather(g + 1, other)

        wait_gather(buf)
        pltpu.async_copy(
            rows_v.at[buf],
            out_hbm.at[pl.ds(tok_base + g * TOKENS_PER_GROUP,
                             TOKENS_PER_GROUP)],
            ssem[buf])

    wait_scatter(0)
    wait_scatter(1)

  return gather_kernel


def kernel(var_indices, var_embedding):
  n, s = var_indices.shape
  idx = var_indices.astype(jnp.int32)
  return _make_gather(n, s)(idx, var_embedding)


# speedup vs baseline: 7.1750x; 1.4558x over previous
import functools

import jax
import jax.numpy as jnp
from jax import lax
from jax.experimental import pallas as pl
from jax.experimental.pallas import tpu as pltpu
from jax.experimental.pallas import tpu_sc as plsc

D_MODEL = 128
NUM_CORES = 2
NUM_SUBCORES = 16
NUM_WORKERS = NUM_CORES * NUM_SUBCORES

TOKENS_PER_GROUP = 4


def _make_gather(N: int, S: int, V: int):
  assert N % (NUM_WORKERS * TOKENS_PER_GROUP) == 0
  t_per_w = N // NUM_WORKERS
  n_groups = t_per_w // TOKENS_PER_GROUP
  assert n_groups % 2 == 0

  mesh = plsc.VectorSubcoreMesh(
      core_axis_name="c", subcore_axis_name="s",
      num_cores=NUM_CORES, num_subcores=NUM_SUBCORES)

  @functools.partial(
      pl.kernel,
      out_type=jax.ShapeDtypeStruct((N, S, D_MODEL), jnp.float32),
      mesh=mesh,
      scratch_types=[
          pltpu.VMEM((2, TOKENS_PER_GROUP, S), jnp.int32),
          pltpu.VMEM((2, TOKENS_PER_GROUP, S, D_MODEL), jnp.float32),
          pltpu.VMEM_SHARED((V, D_MODEL), jnp.float32),
          pltpu.SemaphoreType.DMA,
          pltpu.SemaphoreType.DMA,
          pltpu.SemaphoreType.DMA,
          pltpu.SemaphoreType.DMA,
      ],
  )
  def gather_kernel(idx_hbm, table_hbm, out_hbm, idx_v, rows_v, table_sp,
                    gsem0, gsem1, ssem0, ssem1):
    wid = lax.axis_index("s") * NUM_CORES + lax.axis_index("c")
    tok_base = wid * t_per_w
    gsem = (gsem0, gsem1)
    ssem = (ssem0, ssem1)

    @pl.when(lax.axis_index("s") == 0)
    def _():
      pltpu.sync_copy(table_hbm, table_sp)

    plsc.subcore_barrier()

    def fire_gather(g, buf):
      tok0 = tok_base + g * TOKENS_PER_GROUP
      pltpu.sync_copy(idx_hbm.at[pl.ds(tok0, TOKENS_PER_GROUP)],
                      idx_v.at[buf])
      for j in range(TOKENS_PER_GROUP):
        pltpu.async_copy(
            table_sp.at[idx_v.at[buf].at[j]],
            rows_v.at[buf].at[j],
            gsem[buf])

    def wait_gather(buf):
      for j in range(TOKENS_PER_GROUP):
        pltpu.make_async_copy(
            table_sp.at[idx_v.at[buf].at[j]],
            rows_v.at[buf].at[j],
            gsem[buf]).wait()

    def wait_scatter(buf):
      pltpu.make_async_copy(
          rows_v.at[buf], out_hbm.at[pl.ds(tok_base, TOKENS_PER_GROUP)],
          ssem[buf]).wait()

    fire_gather(0, 0)

    @pl.loop(0, n_groups // 2)
    def _(p):
      for buf in (0, 1):
        g = 2 * p + buf
        other = 1 - buf

        @pl.when(g + 1 < n_groups)
        def _():
          @pl.when(g >= 1)
          def _():
            wait_scatter(other)
          fire_gather(g + 1, other)

        wait_gather(buf)
        pltpu.async_copy(
            rows_v.at[buf],
            out_hbm.at[pl.ds(tok_base + g * TOKENS_PER_GROUP,
                             TOKENS_PER_GROUP)],
            ssem[buf])

    wait_scatter(0)
    wait_scatter(1)

  return gather_kernel


def kernel(var_indices, var_embedding):
  n, s = var_indices.shape
  idx = var_indices.astype(jnp.int32)
  return _make_gather(n, s, var_embedding.shape[0])(idx, var_embedding)
